# Initial kernel scaffold; baseline (speedup 1.0000x reference)
import jax
import jax.numpy as jnp
from jax.experimental import pallas as pl


def kernel(atom_feature, edge_vector, edge_index, W0, W1):
    raise NotImplementedError("write your pallas kernel here")


# speedup vs baseline: 2.2385x; 2.2385x over previous
import functools

import jax
import jax.numpy as jnp
from jax import lax
from jax.experimental import pallas as pl
from jax.experimental.pallas import tpu as pltpu
from jax.experimental.pallas import tpu_sc as plsc

_NC = 2
_NS = 16
_NW = _NC * _NS

_N = 10000
_E = 320000
_CIN = 128
_C1 = 32
_V1 = 3 * _C1
_INV = 1.0 / (_CIN ** 0.5)
_SQRT3 = 3.0 ** 0.5

_EPW = _E // _NW
_CB = 80
_NCHUNK = _EPW // _CB
_FCH = 80
_NFCH = _N // _FCH
_FR = (_NFCH + _NS - 1) // _NS


def _bmat_body(a_ref, w1_ref, b_ref):
    b_ref[...] = jnp.dot(a_ref[...], w1_ref[...],
                         preferred_element_type=jnp.float32) * _INV


def _sh_body(v_ref, o_ref):
    v = v_ref[...]
    n2 = jnp.sum(v * v, axis=1, keepdims=True)
    norm = jnp.sqrt(n2)
    o_ref[...] = _SQRT3 * v / jnp.maximum(norm, 1e-12)


def _final_body(p0_ref, p1_ref, a_ref, w0_ref, perm_ref, o0_ref, o1_ref):
    s0 = p0_ref[0] + p0_ref[1]
    o0_ref[...] = jnp.dot(s0, w0_ref[...],
                          preferred_element_type=jnp.float32) * _INV + a_ref[...]
    o1_ref[...] = jnp.dot(p1_ref[0] + p1_ref[1], perm_ref[...],
                          preferred_element_type=jnp.float32)


def _zero_stage(stage, rows, width):
    nv = width // 16
    def zrow(i, _):
        for j in range(nv):
            stage[i, 16 * j:16 * (j + 1)] = jnp.zeros((16,), jnp.float32)
        return _
    lax.fori_loop(0, rows, zrow, None)


def _zero_acc(sid, zbuf, acc):
    for r in range(_FR):
        idx = sid + _NS * r
        def _do(idx=idx):
            pltpu.sync_copy(zbuf, acc.at[pl.ds(idx * _FCH, _FCH)])
        if (r + 1) * _NS <= _NFCH:
            _do()
        else:
            pl.when(idx < _NFCH)(_do)


def _flush_acc(sid, cid, acc, out_hbm):
    for r in range(_FR):
        idx = sid + _NS * r
        def _do(idx=idx):
            pltpu.sync_copy(acc.at[pl.ds(idx * _FCH, _FCH)],
                            out_hbm.at[cid, pl.ds(idx * _FCH, _FCH)])
        if (r + 1) * _NS <= _NFCH:
            _do()
        else:
            pl.when(idx < _NFCH)(_do)


def _seg0_body(a_hbm, src_hbm, dst_hbm, p0_hbm, sidx, didx, rows, acc, sem):
    cid = lax.axis_index("c")
    sid = lax.axis_index("s")
    wid = cid * _NS + sid

    _zero_stage(rows, _CB, _CIN)
    _zero_acc(sid, rows, acc)
    plsc.subcore_barrier()

    e0 = wid * _EPW

    def step(j, carry):
        base = e0 + j * _CB
        pltpu.sync_copy(src_hbm.at[pl.ds(base, _CB)], sidx)
        pltpu.sync_copy(dst_hbm.at[pl.ds(base, _CB)], didx)
        pltpu.async_copy(a_hbm.at[sidx], rows, sem).wait()
        pltpu.sync_copy(rows, acc.at[didx], add=True)
        return carry

    lax.fori_loop(0, _NCHUNK, step, 0)
    plsc.subcore_barrier()

    _flush_acc(sid, cid, acc, p0_hbm)


def _seg1_body(b_hbm, src_hbm, dst_hbm, sh_hbm, p1_hbm, sidx, didx, shf,
               rows, scaled, acc, sem):
    cid = lax.axis_index("c")
    sid = lax.axis_index("s")
    wid = cid * _NS + sid

    _zero_stage(scaled, _CB, _CIN)
    _zero_acc(sid, scaled, acc)
    plsc.subcore_barrier()

    e0 = wid * _EPW

    def step(j, carry):
        base = e0 + j * _CB
        pltpu.sync_copy(src_hbm.at[pl.ds(base, _CB)], sidx)
        pltpu.sync_copy(dst_hbm.at[pl.ds(base, _CB)], didx)
        pltpu.sync_copy(sh_hbm.at[pl.ds(base * 4, _CB * 4)],
                        shf.at[pl.ds(0, _CB * 4)])
        pltpu.async_copy(b_hbm.at[sidx], rows, sem).wait()

        def edge(e, c2):
            r0 = rows[e, 0:16]
            r1 = rows[e, 16:32]
            sv = shf[pl.ds(4 * e, 16)]
            for k in range(3):
                s = sv[k]
                scaled[e, 32 * k:32 * k + 16] = r0 * s
                scaled[e, 32 * k + 16:32 * k + 32] = r1 * s
            return c2

        lax.fori_loop(0, _CB, edge, 0)
        pltpu.sync_copy(scaled, acc.at[didx], add=True)
        return carry

    lax.fori_loop(0, _NCHUNK, step, 0)
    plsc.subcore_barrier()

    _flush_acc(sid, cid, acc, p1_hbm)


@jax.jit
def kernel(atom_feature, edge_vector, edge_index, W0, W1):
    src = edge_index[0]
    dst = edge_index[1]
    ev4 = jnp.concatenate(
        [edge_vector, jnp.zeros((_E, 1), jnp.float32)], axis=1)

    B = pl.pallas_call(
        _bmat_body,
        grid=(10,),
        in_specs=[
            pl.BlockSpec((_N // 10, _CIN), lambda i: (i, 0)),
            pl.BlockSpec((_CIN, _CIN), lambda i: (0, 0)),
        ],
        out_specs=pl.BlockSpec((_N // 10, _CIN), lambda i: (i, 0)),
        out_shape=jax.ShapeDtypeStruct((_N, _CIN), jnp.float32),
    )(atom_feature, jnp.pad(W1, ((0, 0), (0, _CIN - _C1))))

    sh = pl.pallas_call(
        _sh_body,
        grid=(160,),
        in_specs=[pl.BlockSpec((_E // 160, 4), lambda i: (i, 0))],
        out_specs=pl.BlockSpec((_E // 160, 4), lambda i: (i, 0)),
        out_shape=jax.ShapeDtypeStruct((_E, 4), jnp.float32),
    )(ev4)

    mesh = plsc.VectorSubcoreMesh(core_axis_name="c", subcore_axis_name="s")

    p0 = pl.kernel(
        _seg0_body,
        out_type=jax.ShapeDtypeStruct((_NC, _N, _CIN), jnp.float32),
        mesh=mesh,
        compiler_params=pltpu.CompilerParams(use_tc_tiling_on_sc=True),
        scratch_types=[
            pltpu.VMEM((_CB,), jnp.int32),
            pltpu.VMEM((_CB,), jnp.int32),
            pltpu.VMEM((_CB, _CIN), jnp.float32),
            pltpu.VMEM_SHARED((_N, _CIN), jnp.float32),
            pltpu.SemaphoreType.DMA,
        ],
    )(atom_feature, src, dst)

    p1 = pl.kernel(
        _seg1_body,
        out_type=jax.ShapeDtypeStruct((_NC, _N, _CIN), jnp.float32),
        mesh=mesh,
        compiler_params=pltpu.CompilerParams(use_tc_tiling_on_sc=True),
        scratch_types=[
            pltpu.VMEM((_CB,), jnp.int32),
            pltpu.VMEM((_CB,), jnp.int32),
            pltpu.VMEM((_CB * 4 + 16,), jnp.float32),
            pltpu.VMEM((_CB, _CIN), jnp.float32),
            pltpu.VMEM((_CB, _CIN), jnp.float32),
            pltpu.VMEM_SHARED((_N, _CIN), jnp.float32),
            pltpu.SemaphoreType.DMA,
        ],
    )(B, src, dst, sh.reshape(_E * 4))

    ks = jnp.arange(_V1) % 3
    cs = jnp.arange(_V1) // 3
    perm = jnp.zeros((_CIN, _V1), jnp.float32).at[
        ks * _C1 + cs, jnp.arange(_V1)].set(1.0)

    out0, out1 = pl.pallas_call(
        _final_body,
        grid=(10,),
        in_specs=[
            pl.BlockSpec((_NC, _N // 10, _CIN), lambda i: (0, i, 0)),
            pl.BlockSpec((_NC, _N // 10, _CIN), lambda i: (0, i, 0)),
            pl.BlockSpec((_N // 10, _CIN), lambda i: (i, 0)),
            pl.BlockSpec((_CIN, _CIN), lambda i: (0, 0)),
            pl.BlockSpec((_CIN, _V1), lambda i: (0, 0)),
        ],
        out_specs=[
            pl.BlockSpec((_N // 10, _CIN), lambda i: (i, 0)),
            pl.BlockSpec((_N // 10, _V1), lambda i: (i, 0)),
        ],
        out_shape=[
            jax.ShapeDtypeStruct((_N, _CIN), jnp.float32),
            jax.ShapeDtypeStruct((_N, _V1), jnp.float32),
        ],
    )(p0, p1, atom_feature, W0, perm)

    return jnp.concatenate([out0, out1], axis=1)


# speedup vs baseline: 2.8657x; 1.2802x over previous
import jax
import jax.numpy as jnp
from jax import lax
from jax.experimental import pallas as pl
from jax.experimental.pallas import tpu as pltpu
from jax.experimental.pallas import tpu_sc as plsc

_NC = 2
_NS = 16
_NW = _NC * _NS

_N = 10000
_E = 320000
_CIN = 128
_C1 = 32
_V1 = 3 * _C1
_INV = 1.0 / (_CIN ** 0.5)
_SQRT3 = 3.0 ** 0.5

_EPW = _E // _NW
_CB = 80
_NCHUNK = _EPW // _CB
_FCH = 80
_NFCH = _N // _FCH
_FR = (_NFCH + _NS - 1) // _NS


def _bmat_body(a_ref, w1_ref, b_ref):
    b_ref[...] = jnp.dot(a_ref[...], w1_ref[...],
                         preferred_element_type=jnp.float32) * _INV


def _sh_body(v_ref, o_ref):
    v = v_ref[...]
    n2 = jnp.sum(v * v, axis=1, keepdims=True)
    norm = jnp.sqrt(n2)
    o_ref[...] = _SQRT3 * v / jnp.maximum(norm, 1e-12)


def _final_body(p0_ref, p1_ref, a_ref, w0_ref, perm_ref, o0_ref, o1_ref):
    s0 = p0_ref[0] + p0_ref[1]
    o0_ref[...] = jnp.dot(s0, w0_ref[...],
                          preferred_element_type=jnp.float32) * _INV + a_ref[...]
    o1_ref[...] = jnp.dot(p1_ref[0] + p1_ref[1], perm_ref[...],
                          preferred_element_type=jnp.float32)


def _zero_buf(buf, rows):
    def zrow(i, _):
        for j in range(_CIN // 16):
            buf[i, 16 * j:16 * (j + 1)] = jnp.zeros((16,), jnp.float32)
        return _
    lax.fori_loop(0, rows, zrow, None)


def _zero_acc(sid, zbuf, acc):
    for r in range(_FR):
        idx = sid + _NS * r
        def _do(idx=idx):
            pltpu.sync_copy(zbuf, acc.at[pl.ds(idx * _FCH, _FCH)])
        if (r + 1) * _NS <= _NFCH:
            _do()
        else:
            pl.when(idx < _NFCH)(_do)


def _flush_acc(sid, cid, acc, out_hbm):
    for r in range(_FR):
        idx = sid + _NS * r
        def _do(idx=idx):
            pltpu.sync_copy(acc.at[pl.ds(idx * _FCH, _FCH)],
                            out_hbm.at[cid, pl.ds(idx * _FCH, _FCH)])
        if (r + 1) * _NS <= _NFCH:
            _do()
        else:
            pl.when(idx < _NFCH)(_do)


def _seg_body(a_hbm, b_hbm, src_hbm, dst_hbm, sh_hbm, p0_hbm, p1_hbm,
              sidx0, sidx1, didx0, didx1, rows0, rows1, scaled0, scaled1,
              shf0, shf1, acc, sem0, sem1):
    cid = lax.axis_index("c")
    sid = lax.axis_index("s")
    wid = cid * _NS + sid
    e0 = wid * _EPW

    if True:
        sidx = [sidx0, sidx1]
        didx = [didx0, didx1]
        rows = [rows0, rows1]
        scaled = [scaled0, scaled1]
        shf = [shf0, shf1]
        sems = [sem0, sem1]

        def load_idx(jc, b):
            base = e0 + jc * _CB
            pltpu.sync_copy(src_hbm.at[pl.ds(base, _CB)], sidx[b])
            pltpu.sync_copy(dst_hbm.at[pl.ds(base, _CB)], didx[b])

        _zero_buf(scaled[0], _CB)
        _zero_acc(sid, scaled[0], acc)
        plsc.subcore_barrier()

        load_idx(0, 0)
        pltpu.async_copy(a_hbm.at[sidx[0]], rows[0], sems[0])

        def stepA(j2, carry):
            for b in range(2):
                jc = 2 * j2 + b
                def pre(jc=jc, b=b):
                    load_idx(jc + 1, 1 - b)
                    pltpu.async_copy(a_hbm.at[sidx[1 - b]],
                                     rows[1 - b], sems[1 - b])
                pre()
                pltpu.make_async_copy(a_hbm.at[sidx[b]], rows[b],
                                      sems[b]).wait()
                pltpu.sync_copy(rows[b], acc.at[didx[b]], add=True)
            return carry

        lax.fori_loop(0, (_NCHUNK - 1) // 2, stepA, 0)
        pltpu.make_async_copy(a_hbm.at[sidx[0]], rows[0],
                              sems[0]).wait()
        pltpu.sync_copy(rows[0], acc.at[didx[0]], add=True)

        plsc.subcore_barrier()
        _flush_acc(sid, cid, acc, p0_hbm)
        plsc.subcore_barrier()

        _zero_buf(scaled[0], _CB)
        _zero_buf(scaled[1], _CB)
        _zero_acc(sid, scaled[0], acc)
        plsc.subcore_barrier()

        def load_sh(jc, b):
            base = (e0 + jc * _CB) * 4
            pltpu.sync_copy(sh_hbm.at[pl.ds(base, _CB * 4)],
                            shf[b].at[pl.ds(0, _CB * 4)])

        load_idx(0, 0)
        load_sh(0, 0)
        pltpu.async_copy(b_hbm.at[sidx[0]], rows[0], sems[0])

        def compute(b):
            rb = rows[b]
            sb = scaled[b]
            def grp(g, carry):
                sv = shf[b][pl.ds(16 * g, 16)]
                for i in range(4):
                    e = 4 * g + i
                    r0 = rb[e, 0:16]
                    r1 = rb[e, 16:32]
                    for k in range(3):
                        s = sv[4 * i + k]
                        sb[e, 32 * k:32 * k + 16] = r0 * s
                        sb[e, 32 * k + 16:32 * k + 32] = r1 * s
                return carry
            lax.fori_loop(0, _CB // 4, grp, 0)

        def stepB(j2, carry):
            for b in range(2):
                jc = 2 * j2 + b
                def pre(jc=jc, b=b):
                    load_idx(jc + 1, 1 - b)
                    load_sh(jc + 1, 1 - b)
                    pltpu.async_copy(b_hbm.at[sidx[1 - b]],
                                     rows[1 - b], sems[1 - b])
                pre()
                pltpu.make_async_copy(b_hbm.at[sidx[b]], rows[b],
                                      sems[b]).wait()
                compute(b)
                pltpu.sync_copy(scaled[b], acc.at[didx[b]], add=True)
            return carry

        lax.fori_loop(0, (_NCHUNK - 1) // 2, stepB, 0)
        pltpu.make_async_copy(b_hbm.at[sidx[0]], rows[0],
                              sems[0]).wait()
        compute(0)
        pltpu.sync_copy(scaled[0], acc.at[didx[0]], add=True)

        plsc.subcore_barrier()
        _flush_acc(sid, cid, acc, p1_hbm)


@jax.jit
def kernel(atom_feature, edge_vector, edge_index, W0, W1):
    src = edge_index[0]
    dst = edge_index[1]
    ev4 = jnp.concatenate(
        [edge_vector, jnp.zeros((_E, 1), jnp.float32)], axis=1)

    B = pl.pallas_call(
        _bmat_body,
        grid=(10,),
        in_specs=[
            pl.BlockSpec((_N // 10, _CIN), lambda i: (i, 0)),
            pl.BlockSpec((_CIN, _CIN), lambda i: (0, 0)),
        ],
        out_specs=pl.BlockSpec((_N // 10, _CIN), lambda i: (i, 0)),
        out_shape=jax.ShapeDtypeStruct((_N, _CIN), jnp.float32),
    )(atom_feature, jnp.pad(W1, ((0, 0), (0, _CIN - _C1))))

    sh = pl.pallas_call(
        _sh_body,
        grid=(160,),
        in_specs=[pl.BlockSpec((_E // 160, 4), lambda i: (i, 0))],
        out_specs=pl.BlockSpec((_E // 160, 4), lambda i: (i, 0)),
        out_shape=jax.ShapeDtypeStruct((_E, 4), jnp.float32),
    )(ev4)

    mesh = plsc.VectorSubcoreMesh(core_axis_name="c", subcore_axis_name="s")

    p0, p1 = pl.kernel(
        _seg_body,
        out_type=[
            jax.ShapeDtypeStruct((_NC, _N, _CIN), jnp.float32),
            jax.ShapeDtypeStruct((_NC, _N, _CIN), jnp.float32),
        ],
        mesh=mesh,
        compiler_params=pltpu.CompilerParams(use_tc_tiling_on_sc=True),
        scratch_types=[
            pltpu.VMEM((_CB,), jnp.int32),
            pltpu.VMEM((_CB,), jnp.int32),
            pltpu.VMEM((_CB,), jnp.int32),
            pltpu.VMEM((_CB,), jnp.int32),
            pltpu.VMEM((_CB, _CIN), jnp.float32),
            pltpu.VMEM((_CB, _CIN), jnp.float32),
            pltpu.VMEM((_CB, _CIN), jnp.float32),
            pltpu.VMEM((_CB, _CIN), jnp.float32),
            pltpu.VMEM((_CB * 4 + 16,), jnp.float32),
            pltpu.VMEM((_CB * 4 + 16,), jnp.float32),
            pltpu.VMEM_SHARED((_N, _CIN), jnp.float32),
            pltpu.SemaphoreType.DMA,
            pltpu.SemaphoreType.DMA,
        ],
    )(atom_feature, B, src, dst, sh.reshape(_E * 4))

    ks = jnp.arange(_V1) % 3
    cs = jnp.arange(_V1) // 3
    perm = jnp.zeros((_CIN, _V1), jnp.float32).at[
        ks * _C1 + cs, jnp.arange(_V1)].set(1.0)

    out0, out1 = pl.pallas_call(
        _final_body,
        grid=(10,),
        in_specs=[
            pl.BlockSpec((_NC, _N // 10, _CIN), lambda i: (0, i, 0)),
            pl.BlockSpec((_NC, _N // 10, _CIN), lambda i: (0, i, 0)),
            pl.BlockSpec((_N // 10, _CIN), lambda i: (i, 0)),
            pl.BlockSpec((_CIN, _CIN), lambda i: (0, 0)),
            pl.BlockSpec((_CIN, _V1), lambda i: (0, 0)),
        ],
        out_specs=[
            pl.BlockSpec((_N // 10, _CIN), lambda i: (i, 0)),
            pl.BlockSpec((_N // 10, _V1), lambda i: (i, 0)),
        ],
        out_shape=[
            jax.ShapeDtypeStruct((_N, _CIN), jnp.float32),
            jax.ShapeDtypeStruct((_N, _V1), jnp.float32),
        ],
    )(p0, p1, atom_feature, W0, perm)

    return jnp.concatenate([out0, out1], axis=1)


# speedup vs baseline: 3.4153x; 1.1918x over previous
import jax
import jax.numpy as jnp
from jax import lax
from jax.experimental import pallas as pl
from jax.experimental.pallas import tpu as pltpu
from jax.experimental.pallas import tpu_sc as plsc

_NC = 2
_NS = 16
_NW = _NC * _NS

_N = 10000
_E = 320000
_CIN = 128
_C1 = 32
_V1 = 3 * _C1
_INV = 1.0 / (_CIN ** 0.5)
_SQRT3 = 3.0 ** 0.5

_EPW = _E // _NW
_CB = 80
_NCHUNK = _EPW // _CB
_FCH = 80
_NFCH = _N // _FCH
_FR = (_NFCH + _NS - 1) // _NS


def _bmat_body(a_ref, w1_ref, b_ref):
    b_ref[...] = jnp.dot(a_ref[...], w1_ref[...],
                         preferred_element_type=jnp.float32) * _INV


def _sh_body(v_ref, g_ref, o_ref):
    v = v_ref[...]
    n2 = jnp.dot(v * v, g_ref[...], preferred_element_type=jnp.float32)
    norm = jnp.sqrt(n2)
    o_ref[...] = _SQRT3 * v / jnp.maximum(norm, 1e-12)


def _final_body(p0_ref, p1_ref, a_ref, w0_ref, perm_ref, o0_ref, o1_ref):
    s0 = p0_ref[0] + p0_ref[1]
    o0_ref[...] = jnp.dot(s0, w0_ref[...],
                          preferred_element_type=jnp.float32) * _INV + a_ref[...]
    o1_ref[...] = jnp.dot(p1_ref[0] + p1_ref[1], perm_ref[...],
                          preferred_element_type=jnp.float32)


def _zero_buf(buf, rows):
    def zrow(i, _):
        for j in range(_CIN // 16):
            buf[i, 16 * j:16 * (j + 1)] = jnp.zeros((16,), jnp.float32)
        return _
    lax.fori_loop(0, rows, zrow, None)


def _zero_acc(sid, zbuf, acc):
    for r in range(_FR):
        idx = sid + _NS * r
        def _do(idx=idx):
            pltpu.sync_copy(zbuf, acc.at[pl.ds(idx * _FCH, _FCH)])
        if (r + 1) * _NS <= _NFCH:
            _do()
        else:
            pl.when(idx < _NFCH)(_do)


def _flush_acc(sid, cid, acc, out_hbm):
    for r in range(_FR):
        idx = sid + _NS * r
        def _do(idx=idx):
            pltpu.sync_copy(acc.at[pl.ds(idx * _FCH, _FCH)],
                            out_hbm.at[cid, pl.ds(idx * _FCH, _FCH)])
        if (r + 1) * _NS <= _NFCH:
            _do()
        else:
            pl.when(idx < _NFCH)(_do)


def _seg_body(a_hbm, b_hbm, src_hbm, dst_hbm, sh_hbm, p0_hbm, p1_hbm,
              sidx0, sidx1, didx0, didx1, rows0, rows1, scaled0, scaled1,
              shf0, shf1, acc, sem0, sem1):
    cid = lax.axis_index("c")
    sid = lax.axis_index("s")
    wid = cid * _NS + sid
    e0 = wid * _EPW

    if True:
        sidx = [sidx0, sidx1]
        didx = [didx0, didx1]
        rows = [rows0, rows1]
        scaled = [scaled0, scaled1]
        shf = [shf0, shf1]
        sems = [sem0, sem1]

        def load_idx(jc, b):
            base = e0 + jc * _CB
            pltpu.sync_copy(src_hbm.at[pl.ds(base, _CB)], sidx[b])
            pltpu.sync_copy(dst_hbm.at[pl.ds(base, _CB)], didx[b])

        _zero_buf(scaled[0], _CB)
        _zero_acc(sid, scaled[0], acc)
        plsc.subcore_barrier()

        load_idx(0, 0)
        pltpu.async_copy(a_hbm.at[sidx[0]], rows[0], sems[0])

        def stepA(j2, carry):
            for b in range(2):
                jc = 2 * j2 + b
                def pre(jc=jc, b=b):
                    load_idx(jc + 1, 1 - b)
                    pltpu.async_copy(a_hbm.at[sidx[1 - b]],
                                     rows[1 - b], sems[1 - b])
                pre()
                pltpu.make_async_copy(a_hbm.at[sidx[b]], rows[b],
                                      sems[b]).wait()
                pltpu.sync_copy(rows[b], acc.at[didx[b]], add=True)
            return carry

        lax.fori_loop(0, (_NCHUNK - 1) // 2, stepA, 0)
        pltpu.make_async_copy(a_hbm.at[sidx[0]], rows[0],
                              sems[0]).wait()
        pltpu.sync_copy(rows[0], acc.at[didx[0]], add=True)

        plsc.subcore_barrier()
        _flush_acc(sid, cid, acc, p0_hbm)
        plsc.subcore_barrier()

        _zero_buf(scaled[0], _CB)
        _zero_buf(scaled[1], _CB)
        _zero_acc(sid, scaled[0], acc)
        plsc.subcore_barrier()

        def load_sh(jc, b):
            base = (e0 + jc * _CB) * 4
            pltpu.sync_copy(sh_hbm.at[pl.ds(base, _CB * 4)],
                            shf[b].at[pl.ds(0, _CB * 4)])

        load_idx(0, 0)
        load_sh(0, 0)
        pltpu.async_copy(b_hbm.at[sidx[0]], rows[0], sems[0])

        def compute(b):
            rb = rows[b]
            sb = scaled[b]
            def grp(g, carry):
                sv = shf[b][pl.ds(16 * g, 16)]
                for i in range(4):
                    e = 4 * g + i
                    r0 = rb[e, 0:16]
                    r1 = rb[e, 16:32]
                    for k in range(3):
                        s = sv[4 * i + k]
                        sb[e, 32 * k:32 * k + 16] = r0 * s
                        sb[e, 32 * k + 16:32 * k + 32] = r1 * s
                return carry
            lax.fori_loop(0, _CB // 4, grp, 0)

        def stepB(j2, carry):
            for b in range(2):
                jc = 2 * j2 + b
                def pre(jc=jc, b=b):
                    load_idx(jc + 1, 1 - b)
                    load_sh(jc + 1, 1 - b)
                    pltpu.async_copy(b_hbm.at[sidx[1 - b]],
                                     rows[1 - b], sems[1 - b])
                pre()
                pltpu.make_async_copy(b_hbm.at[sidx[b]], rows[b],
                                      sems[b]).wait()
                compute(b)
                pltpu.sync_copy(scaled[b], acc.at[didx[b]], add=True)
            return carry

        lax.fori_loop(0, (_NCHUNK - 1) // 2, stepB, 0)
        pltpu.make_async_copy(b_hbm.at[sidx[0]], rows[0],
                              sems[0]).wait()
        compute(0)
        pltpu.sync_copy(scaled[0], acc.at[didx[0]], add=True)

        plsc.subcore_barrier()
        _flush_acc(sid, cid, acc, p1_hbm)


@jax.jit
def kernel(atom_feature, edge_vector, edge_index, W0, W1):
    src = edge_index[0]
    dst = edge_index[1]
    ev4 = jnp.concatenate(
        [edge_vector, jnp.zeros((_E, 1), jnp.float32)], axis=1)
    evf = ev4.reshape(_E * 4 // 128, 128)
    gsum = jnp.kron(jnp.eye(32, dtype=jnp.float32),
                    jnp.ones((4, 4), jnp.float32))

    B = pl.pallas_call(
        _bmat_body,
        grid=(10,),
        in_specs=[
            pl.BlockSpec((_N // 10, _CIN), lambda i: (i, 0)),
            pl.BlockSpec((_CIN, _CIN), lambda i: (0, 0)),
        ],
        out_specs=pl.BlockSpec((_N // 10, _CIN), lambda i: (i, 0)),
        out_shape=jax.ShapeDtypeStruct((_N, _CIN), jnp.float32),
    )(atom_feature, jnp.pad(W1, ((0, 0), (0, _CIN - _C1))))

    nr = _E * 4 // 128
    sh = pl.pallas_call(
        _sh_body,
        grid=(10,),
        in_specs=[
            pl.BlockSpec((nr // 10, 128), lambda i: (i, 0)),
            pl.BlockSpec((128, 128), lambda i: (0, 0)),
        ],
        out_specs=pl.BlockSpec((nr // 10, 128), lambda i: (i, 0)),
        out_shape=jax.ShapeDtypeStruct((nr, 128), jnp.float32),
    )(evf, gsum)

    mesh = plsc.VectorSubcoreMesh(core_axis_name="c", subcore_axis_name="s")

    p0, p1 = pl.kernel(
        _seg_body,
        out_type=[
            jax.ShapeDtypeStruct((_NC, _N, _CIN), jnp.float32),
            jax.ShapeDtypeStruct((_NC, _N, _CIN), jnp.float32),
        ],
        mesh=mesh,
        compiler_params=pltpu.CompilerParams(use_tc_tiling_on_sc=True),
        scratch_types=[
            pltpu.VMEM((_CB,), jnp.int32),
            pltpu.VMEM((_CB,), jnp.int32),
            pltpu.VMEM((_CB,), jnp.int32),
            pltpu.VMEM((_CB,), jnp.int32),
            pltpu.VMEM((_CB, _CIN), jnp.float32),
            pltpu.VMEM((_CB, _CIN), jnp.float32),
            pltpu.VMEM((_CB, _CIN), jnp.float32),
            pltpu.VMEM((_CB, _CIN), jnp.float32),
            pltpu.VMEM((_CB * 4 + 16,), jnp.float32),
            pltpu.VMEM((_CB * 4 + 16,), jnp.float32),
            pltpu.VMEM_SHARED((_N, _CIN), jnp.float32),
            pltpu.SemaphoreType.DMA,
            pltpu.SemaphoreType.DMA,
        ],
    )(atom_feature, B, src, dst, sh.reshape(_E * 4))

    ks = jnp.arange(_V1) % 3
    cs = jnp.arange(_V1) // 3
    perm = jnp.zeros((_CIN, _V1), jnp.float32).at[
        ks * _C1 + cs, jnp.arange(_V1)].set(1.0)

    out0, out1 = pl.pallas_call(
        _final_body,
        grid=(10,),
        in_specs=[
            pl.BlockSpec((_NC, _N // 10, _CIN), lambda i: (0, i, 0)),
            pl.BlockSpec((_NC, _N // 10, _CIN), lambda i: (0, i, 0)),
            pl.BlockSpec((_N // 10, _CIN), lambda i: (i, 0)),
            pl.BlockSpec((_CIN, _CIN), lambda i: (0, 0)),
            pl.BlockSpec((_CIN, _V1), lambda i: (0, 0)),
        ],
        out_specs=[
            pl.BlockSpec((_N // 10, _CIN), lambda i: (i, 0)),
            pl.BlockSpec((_N // 10, _V1), lambda i: (i, 0)),
        ],
        out_shape=[
            jax.ShapeDtypeStruct((_N, _CIN), jnp.float32),
            jax.ShapeDtypeStruct((_N, _V1), jnp.float32),
        ],
    )(p0, p1, atom_feature, W0, perm)

    return jnp.concatenate([out0, out1], axis=1)


# speedup vs baseline: 3.9030x; 1.1428x over previous
import jax
import jax.numpy as jnp
from jax import lax
from jax.experimental import pallas as pl
from jax.experimental.pallas import tpu as pltpu
from jax.experimental.pallas import tpu_sc as plsc

_NC = 2
_NS = 16
_NW = _NC * _NS

_N = 10000
_E = 320000
_CIN = 128
_C1 = 32
_V1 = 3 * _C1
_INV = 1.0 / (_CIN ** 0.5)
_SQRT3 = 3.0 ** 0.5

_EPW = _E // _NW
_CB = 80
_NCHUNK = _EPW // _CB
_FCH = 80
_NFCH = _N // _FCH
_FR = (_NFCH + _NS - 1) // _NS


def _bmat_body(a_ref, w1_ref, b_ref):
    b_ref[...] = jnp.dot(a_ref[...], w1_ref[...],
                         preferred_element_type=jnp.float32) * _INV


def _sh_body(v_ref, g_ref, o_ref):
    v = v_ref[...]
    n2 = jnp.dot(v * v, g_ref[...], preferred_element_type=jnp.float32)
    norm = jnp.sqrt(n2)
    o_ref[...] = _SQRT3 * v / jnp.maximum(norm, 1e-12)


def _final_body(p0_ref, p1_ref, a_ref, w0_ref, perm_ref, o0_ref, o1_ref):
    s0 = p0_ref[0] + p0_ref[1]
    o0_ref[...] = jnp.dot(s0, w0_ref[...],
                          preferred_element_type=jnp.float32) * _INV + a_ref[...]
    o1_ref[...] = jnp.dot(p1_ref[0] + p1_ref[1], perm_ref[...],
                          preferred_element_type=jnp.float32)


def _zero_buf(buf, rows):
    def zrow(i, _):
        for j in range(_CIN // 16):
            buf[i, 16 * j:16 * (j + 1)] = jnp.zeros((16,), jnp.float32)
        return _
    lax.fori_loop(0, rows, zrow, None)


def _zero_acc(sid, zbuf, acc):
    for r in range(_FR):
        idx = sid + _NS * r
        def _do(idx=idx):
            pltpu.sync_copy(zbuf, acc.at[pl.ds(idx * _FCH, _FCH)])
        if (r + 1) * _NS <= _NFCH:
            _do()
        else:
            pl.when(idx < _NFCH)(_do)


def _flush_acc(sid, cid, acc, out_hbm):
    for r in range(_FR):
        idx = sid + _NS * r
        def _do(idx=idx):
            pltpu.sync_copy(acc.at[pl.ds(idx * _FCH, _FCH)],
                            out_hbm.at[cid, pl.ds(idx * _FCH, _FCH)])
        if (r + 1) * _NS <= _NFCH:
            _do()
        else:
            pl.when(idx < _NFCH)(_do)


def _seg_body(a_hbm, b_hbm, src_hbm, dst_hbm, sh_hbm, p0_hbm, p1_hbm,
              sidx0, sidx1, didx0, didx1, rows0, rows1, scaled0, scaled1,
              shf0, shf1, acc, sem0, sem1, semi0, semi1):
    cid = lax.axis_index("c")
    sid = lax.axis_index("s")
    wid = cid * _NS + sid
    e0 = wid * _EPW

    if True:
        sidx = [sidx0, sidx1]
        didx = [didx0, didx1]
        rows = [rows0, rows1]
        scaled = [scaled0, scaled1]
        shf = [shf0, shf1]
        semg = [sem0, sem1]
        semi = [semi0, semi1]

        def start_idx(jc, b, with_sh):
            base = e0 + jc * _CB
            pltpu.async_copy(src_hbm.at[pl.ds(base, _CB)], sidx[b], semi[b])
            pltpu.async_copy(dst_hbm.at[pl.ds(base, _CB)], didx[b], semi[b])
            if with_sh:
                pltpu.async_copy(sh_hbm.at[pl.ds(base * 4, _CB * 4)],
                                 shf[b].at[pl.ds(0, _CB * 4)], semi[b])

        def wait_idx(jc, b, with_sh):
            base = e0 + jc * _CB
            pltpu.make_async_copy(src_hbm.at[pl.ds(base, _CB)], sidx[b],
                                  semi[b]).wait()
            pltpu.make_async_copy(dst_hbm.at[pl.ds(base, _CB)], didx[b],
                                  semi[b]).wait()
            if with_sh:
                pltpu.make_async_copy(sh_hbm.at[pl.ds(base * 4, _CB * 4)],
                                      shf[b].at[pl.ds(0, _CB * 4)],
                                      semi[b]).wait()

        def run_phase(tab_hbm, out_hbm, with_sh, compute):
            start_idx(0, 0, with_sh)
            wait_idx(0, 0, with_sh)
            pltpu.async_copy(tab_hbm.at[sidx[0]], rows[0], semg[0])
            start_idx(1, 1, with_sh)

            def slot(jc, b, first, last2, last):
                if not last:
                    wait_idx(jc + 1, 1 - b, with_sh)
                    pltpu.async_copy(tab_hbm.at[sidx[1 - b]], rows[1 - b],
                                     semg[1 - b])
                pltpu.make_async_copy(tab_hbm.at[sidx[b]], rows[b],
                                      semg[b]).wait()
                if compute is not None:
                    compute(b)
                    pltpu.sync_copy(scaled[b], acc.at[didx[b]], add=True)
                else:
                    pltpu.sync_copy(rows[b], acc.at[didx[b]], add=True)
                if not (last2 or last):
                    start_idx(jc + 2, b, with_sh)

            def step(j2, carry):
                for b in range(2):
                    slot(2 * j2 + b, b, False, False, False)
                return carry

            lax.fori_loop(0, (_NCHUNK - 3) // 2, step, 0)
            slot(_NCHUNK - 3, 0, False, False, False)
            slot(_NCHUNK - 2, 1, False, True, False)
            slot(_NCHUNK - 1, 0, False, True, True)

            plsc.subcore_barrier()
            _flush_acc(sid, cid, acc, out_hbm)

        _zero_buf(scaled[0], _CB)
        _zero_buf(scaled[1], _CB)
        _zero_acc(sid, scaled[0], acc)
        plsc.subcore_barrier()
        run_phase(a_hbm, p0_hbm, False, None)
        plsc.subcore_barrier()

        _zero_acc(sid, scaled[0], acc)
        plsc.subcore_barrier()

        def computeB(b):
            rb = rows[b]
            sb = scaled[b]
            def grp(g, carry):
                sv = shf[b][pl.ds(16 * g, 16)]
                for i in range(4):
                    e = 4 * g + i
                    r0 = rb[e, 0:16]
                    r1 = rb[e, 16:32]
                    for k in range(3):
                        s = sv[4 * i + k]
                        sb[e, 32 * k:32 * k + 16] = r0 * s
                        sb[e, 32 * k + 16:32 * k + 32] = r1 * s
                return carry
            lax.fori_loop(0, _CB // 4, grp, 0, unroll=5)

        run_phase(b_hbm, p1_hbm, True, computeB)


@jax.jit
def kernel(atom_feature, edge_vector, edge_index, W0, W1):
    src = edge_index[0]
    dst = edge_index[1]
    ev4 = jnp.concatenate(
        [edge_vector, jnp.zeros((_E, 1), jnp.float32)], axis=1)
    evf = ev4.reshape(_E * 4 // 128, 128)
    gsum = jnp.kron(jnp.eye(32, dtype=jnp.float32),
                    jnp.ones((4, 4), jnp.float32))

    B = pl.pallas_call(
        _bmat_body,
        grid=(10,),
        in_specs=[
            pl.BlockSpec((_N // 10, _CIN), lambda i: (i, 0)),
            pl.BlockSpec((_CIN, _CIN), lambda i: (0, 0)),
        ],
        out_specs=pl.BlockSpec((_N // 10, _CIN), lambda i: (i, 0)),
        out_shape=jax.ShapeDtypeStruct((_N, _CIN), jnp.float32),
    )(atom_feature, jnp.pad(W1, ((0, 0), (0, _CIN - _C1))))

    nr = _E * 4 // 128
    sh = pl.pallas_call(
        _sh_body,
        grid=(10,),
        in_specs=[
            pl.BlockSpec((nr // 10, 128), lambda i: (i, 0)),
            pl.BlockSpec((128, 128), lambda i: (0, 0)),
        ],
        out_specs=pl.BlockSpec((nr // 10, 128), lambda i: (i, 0)),
        out_shape=jax.ShapeDtypeStruct((nr, 128), jnp.float32),
    )(evf, gsum)

    mesh = plsc.VectorSubcoreMesh(core_axis_name="c", subcore_axis_name="s")

    p0, p1 = pl.kernel(
        _seg_body,
        out_type=[
            jax.ShapeDtypeStruct((_NC, _N, _CIN), jnp.float32),
            jax.ShapeDtypeStruct((_NC, _N, _CIN), jnp.float32),
        ],
        mesh=mesh,
        compiler_params=pltpu.CompilerParams(use_tc_tiling_on_sc=True),
        scratch_types=[
            pltpu.VMEM((_CB,), jnp.int32),
            pltpu.VMEM((_CB,), jnp.int32),
            pltpu.VMEM((_CB,), jnp.int32),
            pltpu.VMEM((_CB,), jnp.int32),
            pltpu.VMEM((_CB, _CIN), jnp.float32),
            pltpu.VMEM((_CB, _CIN), jnp.float32),
            pltpu.VMEM((_CB, _CIN), jnp.float32),
            pltpu.VMEM((_CB, _CIN), jnp.float32),
            pltpu.VMEM((_CB * 4 + 16,), jnp.float32),
            pltpu.VMEM((_CB * 4 + 16,), jnp.float32),
            pltpu.VMEM_SHARED((_N, _CIN), jnp.float32),
            pltpu.SemaphoreType.DMA,
            pltpu.SemaphoreType.DMA,
            pltpu.SemaphoreType.DMA,
            pltpu.SemaphoreType.DMA,
        ],
    )(atom_feature, B, src, dst, sh.reshape(_E * 4))

    ks = jnp.arange(_V1) % 3
    cs = jnp.arange(_V1) // 3
    perm = jnp.zeros((_CIN, _V1), jnp.float32).at[
        ks * _C1 + cs, jnp.arange(_V1)].set(1.0)

    out0, out1 = pl.pallas_call(
        _final_body,
        grid=(10,),
        in_specs=[
            pl.BlockSpec((_NC, _N // 10, _CIN), lambda i: (0, i, 0)),
            pl.BlockSpec((_NC, _N // 10, _CIN), lambda i: (0, i, 0)),
            pl.BlockSpec((_N // 10, _CIN), lambda i: (i, 0)),
            pl.BlockSpec((_CIN, _CIN), lambda i: (0, 0)),
            pl.BlockSpec((_CIN, _V1), lambda i: (0, 0)),
        ],
        out_specs=[
            pl.BlockSpec((_N // 10, _CIN), lambda i: (i, 0)),
            pl.BlockSpec((_N // 10, _V1), lambda i: (i, 0)),
        ],
        out_shape=[
            jax.ShapeDtypeStruct((_N, _CIN), jnp.float32),
            jax.ShapeDtypeStruct((_N, _V1), jnp.float32),
        ],
    )(p0, p1, atom_feature, W0, perm)

    return jnp.concatenate([out0, out1], axis=1)


# speedup vs baseline: 6.0701x; 1.5553x over previous
import jax
import jax.numpy as jnp
from jax import lax
from jax.experimental import pallas as pl
from jax.experimental.pallas import tpu as pltpu
from jax.experimental.pallas import tpu_sc as plsc

_NC = 2
_NS = 16
_NW = _NC * _NS

_N = 10000
_E = 320000
_CIN = 128
_C1 = 32
_V1 = 3 * _C1
_INV = 1.0 / (_CIN ** 0.5)
_SQRT3 = 3.0 ** 0.5

_EPW = _E // _NW
_CB = 80
_NCHUNK = _EPW // _CB
_FCH = 80
_NFCH = _N // _FCH
_FR = (_NFCH + _NS - 1) // _NS


def _bmat_body(a_ref, w1_ref, b_ref):
    b_ref[...] = jnp.dot(a_ref[...], w1_ref[...],
                         preferred_element_type=jnp.float32) * _INV


def _sh_body(x_ref, y_ref, z_ref, px_ref, g_ref, o_ref):
    px = px_ref[...]
    v = jnp.dot(x_ref[...], px[:, 0:128],
                preferred_element_type=jnp.float32)
    v += jnp.dot(y_ref[...], px[:, 128:256],
                 preferred_element_type=jnp.float32)
    v += jnp.dot(z_ref[...], px[:, 256:384],
                 preferred_element_type=jnp.float32)
    n2 = jnp.dot(v * v, g_ref[...], preferred_element_type=jnp.float32)
    norm = jnp.sqrt(n2)
    o_ref[...] = _SQRT3 * v / jnp.maximum(norm, 1e-12)


def _final_body(p0_ref, p1_ref, a_ref, w0_ref, perm_ref, o0_ref, o1_ref):
    s0 = p0_ref[0] + p0_ref[1]
    o0_ref[...] = jnp.dot(s0, w0_ref[...],
                          preferred_element_type=jnp.float32) * _INV + a_ref[...]
    o1_ref[...] = jnp.dot(p1_ref[0] + p1_ref[1], perm_ref[...],
                          preferred_element_type=jnp.float32)


def _zero_buf(buf, rows):
    def zrow(i, _):
        for j in range(_CIN // 16):
            buf[i, 16 * j:16 * (j + 1)] = jnp.zeros((16,), jnp.float32)
        return _
    lax.fori_loop(0, rows, zrow, None)


def _zero_acc(sid, zbuf, acc):
    for r in range(_FR):
        idx = sid + _NS * r
        def _do(idx=idx):
            pltpu.sync_copy(zbuf, acc.at[pl.ds(idx * _FCH, _FCH)])
        if (r + 1) * _NS <= _NFCH:
            _do()
        else:
            pl.when(idx < _NFCH)(_do)


def _flush_acc(sid, cid, acc, out_hbm):
    for r in range(_FR):
        idx = sid + _NS * r
        def _do(idx=idx):
            pltpu.sync_copy(acc.at[pl.ds(idx * _FCH, _FCH)],
                            out_hbm.at[cid, pl.ds(idx * _FCH, _FCH)])
        if (r + 1) * _NS <= _NFCH:
            _do()
        else:
            pl.when(idx < _NFCH)(_do)


def _seg_body(a_hbm, b_hbm, src_hbm, dst_hbm, sh_hbm, p0_hbm, p1_hbm,
              sidx0, sidx1, didx0, didx1, rows0, rows1, scaled0, scaled1,
              shf0, shf1, acc, sem0, sem1, semi0, semi1):
    cid = lax.axis_index("c")
    sid = lax.axis_index("s")
    wid = cid * _NS + sid
    e0 = wid * _EPW

    if True:
        sidx = [sidx0, sidx1]
        didx = [didx0, didx1]
        rows = [rows0, rows1]
        scaled = [scaled0, scaled1]
        shf = [shf0, shf1]
        semg = [sem0, sem1]
        semi = [semi0, semi1]

        def start_idx(jc, b, with_sh):
            base = e0 + jc * _CB
            pltpu.async_copy(src_hbm.at[pl.ds(base, _CB)], sidx[b], semi[b])
            pltpu.async_copy(dst_hbm.at[pl.ds(base, _CB)], didx[b], semi[b])
            if with_sh:
                pltpu.async_copy(sh_hbm.at[pl.ds(base * 4, _CB * 4)],
                                 shf[b].at[pl.ds(0, _CB * 4)], semi[b])

        def wait_idx(jc, b, with_sh):
            base = e0 + jc * _CB
            pltpu.make_async_copy(src_hbm.at[pl.ds(base, _CB)], sidx[b],
                                  semi[b]).wait()
            pltpu.make_async_copy(dst_hbm.at[pl.ds(base, _CB)], didx[b],
                                  semi[b]).wait()
            if with_sh:
                pltpu.make_async_copy(sh_hbm.at[pl.ds(base * 4, _CB * 4)],
                                      shf[b].at[pl.ds(0, _CB * 4)],
                                      semi[b]).wait()

        def run_phase(tab_hbm, out_hbm, with_sh, compute):
            start_idx(0, 0, with_sh)
            wait_idx(0, 0, with_sh)
            pltpu.async_copy(tab_hbm.at[sidx[0]], rows[0], semg[0])
            start_idx(1, 1, with_sh)

            def slot(jc, b, first, last2, last):
                if not last:
                    wait_idx(jc + 1, 1 - b, with_sh)
                    pltpu.async_copy(tab_hbm.at[sidx[1 - b]], rows[1 - b],
                                     semg[1 - b])
                pltpu.make_async_copy(tab_hbm.at[sidx[b]], rows[b],
                                      semg[b]).wait()
                if compute is not None:
                    compute(b)
                    pltpu.sync_copy(scaled[b], acc.at[didx[b]], add=True)
                else:
                    pltpu.sync_copy(rows[b], acc.at[didx[b]], add=True)
                if not (last2 or last):
                    start_idx(jc + 2, b, with_sh)

            def step(j2, carry):
                for b in range(2):
                    slot(2 * j2 + b, b, False, False, False)
                return carry

            lax.fori_loop(0, (_NCHUNK - 3) // 2, step, 0)
            slot(_NCHUNK - 3, 0, False, False, False)
            slot(_NCHUNK - 2, 1, False, True, False)
            slot(_NCHUNK - 1, 0, False, True, True)

            plsc.subcore_barrier()
            _flush_acc(sid, cid, acc, out_hbm)

        _zero_buf(scaled[0], _CB)
        _zero_buf(scaled[1], _CB)
        _zero_acc(sid, scaled[0], acc)
        plsc.subcore_barrier()
        run_phase(a_hbm, p0_hbm, False, None)
        plsc.subcore_barrier()

        _zero_acc(sid, scaled[0], acc)
        plsc.subcore_barrier()

        def computeB(b):
            rb = rows[b]
            sb = scaled[b]
            def grp(g, carry):
                sv = shf[b][pl.ds(16 * g, 16)]
                for i in range(4):
                    e = 4 * g + i
                    r0 = rb[e, 0:16]
                    r1 = rb[e, 16:32]
                    for k in range(3):
                        s = sv[4 * i + k]
                        sb[e, 32 * k:32 * k + 16] = r0 * s
                        sb[e, 32 * k + 16:32 * k + 32] = r1 * s
                return carry
            lax.fori_loop(0, _CB // 4, grp, 0, unroll=5)

        run_phase(b_hbm, p1_hbm, True, computeB)


@jax.jit
def kernel(atom_feature, edge_vector, edge_index, W0, W1):
    src = edge_index[0]
    dst = edge_index[1]
    x2 = edge_vector[:, 0].reshape(_N, 32)
    y2 = edge_vector[:, 1].reshape(_N, 32)
    z2 = edge_vector[:, 2].reshape(_N, 32)
    eye32 = jnp.eye(32, dtype=jnp.float32)
    px = jnp.concatenate(
        [jnp.kron(eye32, jnp.eye(1, 4, k, dtype=jnp.float32))
         for k in range(3)], axis=1)
    gsum = jnp.kron(eye32, jnp.ones((4, 4), jnp.float32))

    B = pl.pallas_call(
        _bmat_body,
        grid=(10,),
        in_specs=[
            pl.BlockSpec((_N // 10, _CIN), lambda i: (i, 0)),
            pl.BlockSpec((_CIN, _CIN), lambda i: (0, 0)),
        ],
        out_specs=pl.BlockSpec((_N // 10, _CIN), lambda i: (i, 0)),
        out_shape=jax.ShapeDtypeStruct((_N, _CIN), jnp.float32),
    )(atom_feature, jnp.pad(W1, ((0, 0), (0, _CIN - _C1))))

    nr = _E * 4 // 128
    sh = pl.pallas_call(
        _sh_body,
        grid=(10,),
        in_specs=[
            pl.BlockSpec((_N // 10, 32), lambda i: (i, 0)),
            pl.BlockSpec((_N // 10, 32), lambda i: (i, 0)),
            pl.BlockSpec((_N // 10, 32), lambda i: (i, 0)),
            pl.BlockSpec((32, 384), lambda i: (0, 0)),
            pl.BlockSpec((128, 128), lambda i: (0, 0)),
        ],
        out_specs=pl.BlockSpec((nr // 10, 128), lambda i: (i, 0)),
        out_shape=jax.ShapeDtypeStruct((nr, 128), jnp.float32),
    )(x2, y2, z2, px, gsum)

    mesh = plsc.VectorSubcoreMesh(core_axis_name="c", subcore_axis_name="s")

    p0, p1 = pl.kernel(
        _seg_body,
        out_type=[
            jax.ShapeDtypeStruct((_NC, _N, _CIN), jnp.float32),
            jax.ShapeDtypeStruct((_NC, _N, _CIN), jnp.float32),
        ],
        mesh=mesh,
        compiler_params=pltpu.CompilerParams(use_tc_tiling_on_sc=True),
        scratch_types=[
            pltpu.VMEM((_CB,), jnp.int32),
            pltpu.VMEM((_CB,), jnp.int32),
            pltpu.VMEM((_CB,), jnp.int32),
            pltpu.VMEM((_CB,), jnp.int32),
            pltpu.VMEM((_CB, _CIN), jnp.float32),
            pltpu.VMEM((_CB, _CIN), jnp.float32),
            pltpu.VMEM((_CB, _CIN), jnp.float32),
            pltpu.VMEM((_CB, _CIN), jnp.float32),
            pltpu.VMEM((_CB * 4 + 16,), jnp.float32),
            pltpu.VMEM((_CB * 4 + 16,), jnp.float32),
            pltpu.VMEM_SHARED((_N, _CIN), jnp.float32),
            pltpu.SemaphoreType.DMA,
            pltpu.SemaphoreType.DMA,
            pltpu.SemaphoreType.DMA,
            pltpu.SemaphoreType.DMA,
        ],
    )(atom_feature, B, src, dst, sh.reshape(_E * 4))

    ks = jnp.arange(_V1) % 3
    cs = jnp.arange(_V1) // 3
    perm = jnp.zeros((_CIN, _V1), jnp.float32).at[
        ks * _C1 + cs, jnp.arange(_V1)].set(1.0)

    out0, out1 = pl.pallas_call(
        _final_body,
        grid=(10,),
        in_specs=[
            pl.BlockSpec((_NC, _N // 10, _CIN), lambda i: (0, i, 0)),
            pl.BlockSpec((_NC, _N // 10, _CIN), lambda i: (0, i, 0)),
            pl.BlockSpec((_N // 10, _CIN), lambda i: (i, 0)),
            pl.BlockSpec((_CIN, _CIN), lambda i: (0, 0)),
            pl.BlockSpec((_CIN, _V1), lambda i: (0, 0)),
        ],
        out_specs=[
            pl.BlockSpec((_N // 10, _CIN), lambda i: (i, 0)),
            pl.BlockSpec((_N // 10, _V1), lambda i: (i, 0)),
        ],
        out_shape=[
            jax.ShapeDtypeStruct((_N, _CIN), jnp.float32),
            jax.ShapeDtypeStruct((_N, _V1), jnp.float32),
        ],
    )(p0, p1, atom_feature, W0, perm)

    return jnp.concatenate([out0, out1], axis=1)


# speedup vs baseline: 7.0621x; 1.1634x over previous
import jax
import jax.numpy as jnp
from jax import lax
from jax.experimental import pallas as pl
from jax.experimental.pallas import tpu as pltpu
from jax.experimental.pallas import tpu_sc as plsc

_NC = 2
_NS = 16
_NW = _NC * _NS

_N = 10000
_E = 320000
_CIN = 128
_C1 = 32
_V1 = 3 * _C1
_INV = 1.0 / (_CIN ** 0.5)
_SQRT3 = 3.0 ** 0.5

_EPW = _E // _NW
_CB = 80
_NCHUNK = _EPW // _CB
_FCH = 80
_NFCH = _N // _FCH
_FR = (_NFCH + _NS - 1) // _NS


def _bmat_body(a_ref, w1_ref, b_ref):
    b_ref[...] = jnp.dot(a_ref[...], w1_ref[...],
                         preferred_element_type=jnp.float32) * _INV


def _sh_body(x_ref, y_ref, z_ref, px_ref, g_ref, o_ref):
    px = px_ref[...]
    v = jnp.dot(x_ref[...], px[:, 0:128],
                preferred_element_type=jnp.float32)
    v += jnp.dot(y_ref[...], px[:, 128:256],
                 preferred_element_type=jnp.float32)
    v += jnp.dot(z_ref[...], px[:, 256:384],
                 preferred_element_type=jnp.float32)
    n2 = jnp.dot(v * v, g_ref[...], preferred_element_type=jnp.float32)
    norm = jnp.sqrt(n2)
    o_ref[...] = _SQRT3 * v / jnp.maximum(norm, 1e-12)


def _final_body(p0_ref, p1_ref, a_ref, w0_ref, perm_ref, o0_ref, o1_ref):
    s0 = p0_ref[0] + p0_ref[1]
    o0_ref[...] = jnp.dot(s0, w0_ref[...],
                          preferred_element_type=jnp.float32) * _INV + a_ref[...]
    o1_ref[...] = jnp.dot(p1_ref[0] + p1_ref[1], perm_ref[...],
                          preferred_element_type=jnp.float32)


def _zero_buf(buf, rows):
    def zrow(i, _):
        for j in range(_CIN // 16):
            buf[i, 16 * j:16 * (j + 1)] = jnp.zeros((16,), jnp.float32)
        return _
    lax.fori_loop(0, rows, zrow, None)


def _zero_acc(sid, zbuf, acc):
    for r in range(_FR):
        idx = sid + _NS * r
        def _do(idx=idx):
            pltpu.sync_copy(zbuf, acc.at[pl.ds(idx * _FCH, _FCH)])
        if (r + 1) * _NS <= _NFCH:
            _do()
        else:
            pl.when(idx < _NFCH)(_do)


def _flush_acc(sid, cid, acc, out_hbm):
    for r in range(_FR):
        idx = sid + _NS * r
        def _do(idx=idx):
            pltpu.sync_copy(acc.at[pl.ds(idx * _FCH, _FCH)],
                            out_hbm.at[cid, pl.ds(idx * _FCH, _FCH)])
        if (r + 1) * _NS <= _NFCH:
            _do()
        else:
            pl.when(idx < _NFCH)(_do)


def _seg_body(a_hbm, b_hbm, src_hbm, dst_hbm, sh_hbm, p0_hbm, p1_hbm,
              sidx0, sidx1, didx0, didx1, didx2, rows0, rows1,
              scaled0, scaled1, shf0, shf1, acc,
              sem0, sem1, semi0, semi1, sems0, sems1):
    cid = lax.axis_index("c")
    sid = lax.axis_index("s")
    wid = cid * _NS + sid
    e0 = wid * _EPW

    if True:
        sidx = [sidx0, sidx1]
        didx3 = [didx0, didx1, didx2]
        rows = [rows0, rows1]
        scaled = [scaled0, scaled1]
        shf = [shf0, shf1]
        semg = [sem0, sem1]
        semi = [semi0, semi1]
        sems = [sems0, sems1]

        def start_idx(jc, b2, b3, with_sh):
            base = e0 + jc * _CB
            pltpu.async_copy(src_hbm.at[pl.ds(base, _CB)], sidx[b2], semi[b2])
            pltpu.async_copy(dst_hbm.at[pl.ds(base, _CB)], didx3[b3],
                             semi[b2])
            if with_sh:
                pltpu.async_copy(sh_hbm.at[pl.ds(base * 4, _CB * 4)],
                                 shf[b2].at[pl.ds(0, _CB * 4)], semi[b2])

        def wait_idx(jc, b2, b3, with_sh):
            base = e0 + jc * _CB
            pltpu.make_async_copy(src_hbm.at[pl.ds(base, _CB)], sidx[b2],
                                  semi[b2]).wait()
            pltpu.make_async_copy(dst_hbm.at[pl.ds(base, _CB)], didx3[b3],
                                  semi[b2]).wait()
            if with_sh:
                pltpu.make_async_copy(sh_hbm.at[pl.ds(base * 4, _CB * 4)],
                                      shf[b2].at[pl.ds(0, _CB * 4)],
                                      semi[b2]).wait()

        def run_phase(tab_hbm, out_hbm, with_sh, compute):
            start_idx(0, 0, 0, with_sh)
            wait_idx(0, 0, 0, with_sh)
            pltpu.async_copy(tab_hbm.at[sidx[0]], rows[0], semg[0])
            start_idx(1, 1, 1, with_sh)

            def src_of(b2):
                return scaled[b2] if compute is not None else rows[b2]

            def slot(jc, b2, b3):
                @pl.when(jc + 1 <= _NCHUNK - 1)
                def _():
                    wait_idx(jc + 1, 1 - b2, (b3 + 1) % 3, with_sh)

                @pl.when((jc >= 1) & (jc + 1 <= _NCHUNK - 1))
                def _():
                    pltpu.make_async_copy(
                        src_of(1 - b2), acc.at[didx3[(b3 + 2) % 3]],
                        sems[1 - b2]).wait()

                @pl.when(jc + 1 <= _NCHUNK - 1)
                def _():
                    pltpu.async_copy(tab_hbm.at[sidx[1 - b2]], rows[1 - b2],
                                     semg[1 - b2])

                @pl.when(jc <= _NCHUNK - 1)
                def _():
                    pltpu.make_async_copy(tab_hbm.at[sidx[b2]], rows[b2],
                                          semg[b2]).wait()
                    if compute is not None:
                        compute(b2)
                    pltpu.async_copy(src_of(b2), acc.at[didx3[b3]],
                                     sems[b2], add=True)

                @pl.when(jc + 2 <= _NCHUNK - 1)
                def _():
                    start_idx(jc + 2, b2, (b3 + 2) % 3, with_sh)

            def step(j6, carry):
                for i in range(6):
                    slot(6 * j6 + i, i % 2, i % 3)
                return carry

            lax.fori_loop(0, (_NCHUNK + 5) // 6, step, 0)

            pltpu.make_async_copy(src_of(1), acc.at[didx3[0]],
                                  sems[1]).wait()
            pltpu.make_async_copy(src_of(0), acc.at[didx3[1]],
                                  sems[0]).wait()

            plsc.subcore_barrier()
            _flush_acc(sid, cid, acc, out_hbm)

        _zero_buf(scaled[0], _CB)
        _zero_buf(scaled[1], _CB)
        _zero_acc(sid, scaled[0], acc)
        plsc.subcore_barrier()
        run_phase(a_hbm, p0_hbm, False, None)
        plsc.subcore_barrier()

        _zero_acc(sid, scaled[0], acc)
        plsc.subcore_barrier()

        def computeB(b):
            rb = rows[b]
            sb = scaled[b]
            def grp(g, carry):
                sv = shf[b][pl.ds(16 * g, 16)]
                for i in range(4):
                    e = 4 * g + i
                    r0 = rb[e, 0:16]
                    r1 = rb[e, 16:32]
                    for k in range(3):
                        s = sv[4 * i + k]
                        sb[e, 32 * k:32 * k + 16] = r0 * s
                        sb[e, 32 * k + 16:32 * k + 32] = r1 * s
                return carry
            lax.fori_loop(0, _CB // 4, grp, 0, unroll=5)

        run_phase(b_hbm, p1_hbm, True, computeB)


@jax.jit
def kernel(atom_feature, edge_vector, edge_index, W0, W1):
    src = edge_index[0]
    dst = edge_index[1]
    x2 = edge_vector[:, 0].reshape(_N, 32)
    y2 = edge_vector[:, 1].reshape(_N, 32)
    z2 = edge_vector[:, 2].reshape(_N, 32)
    eye32 = jnp.eye(32, dtype=jnp.float32)
    px = jnp.concatenate(
        [jnp.kron(eye32, jnp.eye(1, 4, k, dtype=jnp.float32))
         for k in range(3)], axis=1)
    gsum = jnp.kron(eye32, jnp.ones((4, 4), jnp.float32))

    B = pl.pallas_call(
        _bmat_body,
        grid=(10,),
        in_specs=[
            pl.BlockSpec((_N // 10, _CIN), lambda i: (i, 0)),
            pl.BlockSpec((_CIN, _CIN), lambda i: (0, 0)),
        ],
        out_specs=pl.BlockSpec((_N // 10, _CIN), lambda i: (i, 0)),
        out_shape=jax.ShapeDtypeStruct((_N, _CIN), jnp.float32),
    )(atom_feature, jnp.pad(W1, ((0, 0), (0, _CIN - _C1))))

    nr = _E * 4 // 128
    sh = pl.pallas_call(
        _sh_body,
        grid=(10,),
        in_specs=[
            pl.BlockSpec((_N // 10, 32), lambda i: (i, 0)),
            pl.BlockSpec((_N // 10, 32), lambda i: (i, 0)),
            pl.BlockSpec((_N // 10, 32), lambda i: (i, 0)),
            pl.BlockSpec((32, 384), lambda i: (0, 0)),
            pl.BlockSpec((128, 128), lambda i: (0, 0)),
        ],
        out_specs=pl.BlockSpec((nr // 10, 128), lambda i: (i, 0)),
        out_shape=jax.ShapeDtypeStruct((nr, 128), jnp.float32),
    )(x2, y2, z2, px, gsum)

    mesh = plsc.VectorSubcoreMesh(core_axis_name="c", subcore_axis_name="s")

    p0, p1 = pl.kernel(
        _seg_body,
        out_type=[
            jax.ShapeDtypeStruct((_NC, _N, _CIN), jnp.float32),
            jax.ShapeDtypeStruct((_NC, _N, _CIN), jnp.float32),
        ],
        mesh=mesh,
        compiler_params=pltpu.CompilerParams(use_tc_tiling_on_sc=True),
        scratch_types=[
            pltpu.VMEM((_CB,), jnp.int32),
            pltpu.VMEM((_CB,), jnp.int32),
            pltpu.VMEM((_CB,), jnp.int32),
            pltpu.VMEM((_CB,), jnp.int32),
            pltpu.VMEM((_CB,), jnp.int32),
            pltpu.VMEM((_CB, _CIN), jnp.float32),
            pltpu.VMEM((_CB, _CIN), jnp.float32),
            pltpu.VMEM((_CB, _CIN), jnp.float32),
            pltpu.VMEM((_CB, _CIN), jnp.float32),
            pltpu.VMEM((_CB * 4 + 16,), jnp.float32),
            pltpu.VMEM((_CB * 4 + 16,), jnp.float32),
            pltpu.VMEM_SHARED((_N, _CIN), jnp.float32),
            pltpu.SemaphoreType.DMA,
            pltpu.SemaphoreType.DMA,
            pltpu.SemaphoreType.DMA,
            pltpu.SemaphoreType.DMA,
            pltpu.SemaphoreType.DMA,
            pltpu.SemaphoreType.DMA,
        ],
    )(atom_feature, B, src, dst, sh.reshape(_E * 4))

    ks = jnp.arange(_V1) % 3
    cs = jnp.arange(_V1) // 3
    perm = jnp.zeros((_CIN, _V1), jnp.float32).at[
        ks * _C1 + cs, jnp.arange(_V1)].set(1.0)

    out0, out1 = pl.pallas_call(
        _final_body,
        grid=(10,),
        in_specs=[
            pl.BlockSpec((_NC, _N // 10, _CIN), lambda i: (0, i, 0)),
            pl.BlockSpec((_NC, _N // 10, _CIN), lambda i: (0, i, 0)),
            pl.BlockSpec((_N // 10, _CIN), lambda i: (i, 0)),
            pl.BlockSpec((_CIN, _CIN), lambda i: (0, 0)),
            pl.BlockSpec((_CIN, _V1), lambda i: (0, 0)),
        ],
        out_specs=[
            pl.BlockSpec((_N // 10, _CIN), lambda i: (i, 0)),
            pl.BlockSpec((_N // 10, _V1), lambda i: (i, 0)),
        ],
        out_shape=[
            jax.ShapeDtypeStruct((_N, _CIN), jnp.float32),
            jax.ShapeDtypeStruct((_N, _V1), jnp.float32),
        ],
    )(p0, p1, atom_feature, W0, perm)

    return jnp.concatenate([out0, out1], axis=1)


# speedup vs baseline: 7.2621x; 1.0283x over previous
import jax
import jax.numpy as jnp
from jax import lax
from jax.experimental import pallas as pl
from jax.experimental.pallas import tpu as pltpu
from jax.experimental.pallas import tpu_sc as plsc

_NC = 2
_NS = 16
_NW = _NC * _NS

_N = 10000
_E = 320000
_CIN = 128
_C1 = 32
_V1 = 3 * _C1
_INV = 1.0 / (_CIN ** 0.5)
_SQRT3 = 3.0 ** 0.5

_EPW = _E // _NW
_CB = 80
_NCHUNK = _EPW // _CB
_FCH = 80
_NFCH = _N // _FCH
_FR = (_NFCH + _NS - 1) // _NS


def _bmat_body(a_ref, w1_ref, b_ref):
    b_ref[...] = jnp.dot(a_ref[...], w1_ref[...],
                         preferred_element_type=jnp.float32) * _INV


def _sh_body(x_ref, y_ref, z_ref, px_ref, g_ref, o_ref):
    px = px_ref[...]
    v = jnp.dot(x_ref[...], px[:, 0:128],
                preferred_element_type=jnp.float32)
    v += jnp.dot(y_ref[...], px[:, 128:256],
                 preferred_element_type=jnp.float32)
    v += jnp.dot(z_ref[...], px[:, 256:384],
                 preferred_element_type=jnp.float32)
    n2 = jnp.dot(v * v, g_ref[...], preferred_element_type=jnp.float32)
    norm = jnp.sqrt(n2)
    o_ref[...] = _SQRT3 * v / jnp.maximum(norm, 1e-12)


def _final_body(p0_ref, p1_ref, a_ref, w0_ref, perm_ref, o0_ref, o1_ref):
    s0 = p0_ref[0] + p0_ref[1]
    o0_ref[...] = jnp.dot(s0, w0_ref[...],
                          preferred_element_type=jnp.float32) * _INV + a_ref[...]
    o1_ref[...] = jnp.dot(p1_ref[0] + p1_ref[1], perm_ref[...],
                          preferred_element_type=jnp.float32)


def _zero_buf(buf, rows):
    def zrow(i, _):
        for j in range(_CIN // 16):
            buf[i, 16 * j:16 * (j + 1)] = jnp.zeros((16,), jnp.float32)
        return _
    lax.fori_loop(0, rows, zrow, None)


def _zero_acc(sid, zbuf, acc):
    for r in range(_FR):
        idx = sid + _NS * r
        def _do(idx=idx):
            pltpu.sync_copy(zbuf, acc.at[pl.ds(idx * _FCH, _FCH)])
        if (r + 1) * _NS <= _NFCH:
            _do()
        else:
            pl.when(idx < _NFCH)(_do)


def _flush_acc(sid, cid, acc, out_hbm):
    for r in range(_FR):
        idx = sid + _NS * r
        def _do(idx=idx):
            pltpu.sync_copy(acc.at[pl.ds(idx * _FCH, _FCH)],
                            out_hbm.at[cid, pl.ds(idx * _FCH, _FCH)])
        if (r + 1) * _NS <= _NFCH:
            _do()
        else:
            pl.when(idx < _NFCH)(_do)


def _seg_body(a_hbm, b_hbm, src_hbm, dst_hbm, sh_hbm, p0_hbm, p1_hbm,
              sidx0, sidx1, didx0, didx1, didx2, rows0, rows1,
              scaled0, scaled1, shf0, shf1, acc,
              sem0, sem1, semi0, semi1, sems0, sems1):
    cid = lax.axis_index("c")
    sid = lax.axis_index("s")
    wid = cid * _NS + sid
    e0 = wid * _EPW

    if True:
        sidx = [sidx0, sidx1]
        didx3 = [didx0, didx1, didx2]
        rows = [rows0, rows1]
        scaled = [scaled0, scaled1]
        shf = [shf0, shf1]
        semg = [sem0, sem1]
        semi = [semi0, semi1]
        sems = [sems0, sems1]

        def start_idx(jc, b2, b3, with_sh):
            base = e0 + jc * _CB
            pltpu.async_copy(src_hbm.at[pl.ds(base, _CB)], sidx[b2], semi[b2])
            pltpu.async_copy(dst_hbm.at[pl.ds(base, _CB)], didx3[b3],
                             semi[b2])
            if with_sh:
                pltpu.async_copy(sh_hbm.at[pl.ds(base * 4, _CB * 4)],
                                 shf[b2].at[pl.ds(0, _CB * 4)], semi[b2])

        def wait_idx(jc, b2, b3, with_sh):
            base = e0 + jc * _CB
            pltpu.make_async_copy(src_hbm.at[pl.ds(base, _CB)], sidx[b2],
                                  semi[b2]).wait()
            pltpu.make_async_copy(dst_hbm.at[pl.ds(base, _CB)], didx3[b3],
                                  semi[b2]).wait()
            if with_sh:
                pltpu.make_async_copy(sh_hbm.at[pl.ds(base * 4, _CB * 4)],
                                      shf[b2].at[pl.ds(0, _CB * 4)],
                                      semi[b2]).wait()

        def run_phase(tab_hbm, out_hbm, with_sh, compute):
            start_idx(0, 0, 0, with_sh)
            wait_idx(0, 0, 0, with_sh)
            pltpu.async_copy(tab_hbm.at[sidx[0]], rows[0], semg[0])
            start_idx(1, 1, 1, with_sh)

            def src_of(b2):
                return scaled[b2] if compute is not None else rows[b2]

            def slot(jc, b2, b3):
                @pl.when(jc + 1 <= _NCHUNK - 1)
                def _():
                    wait_idx(jc + 1, 1 - b2, (b3 + 1) % 3, with_sh)

                @pl.when((jc >= 1) & (jc + 1 <= _NCHUNK - 1))
                def _():
                    pltpu.make_async_copy(
                        src_of(1 - b2), acc.at[didx3[(b3 + 2) % 3]],
                        sems[1 - b2]).wait()

                @pl.when(jc + 1 <= _NCHUNK - 1)
                def _():
                    pltpu.async_copy(tab_hbm.at[sidx[1 - b2]], rows[1 - b2],
                                     semg[1 - b2])

                @pl.when(jc <= _NCHUNK - 1)
                def _():
                    pltpu.make_async_copy(tab_hbm.at[sidx[b2]], rows[b2],
                                          semg[b2]).wait()
                    if compute is not None:
                        compute(b2)
                    pltpu.async_copy(src_of(b2), acc.at[didx3[b3]],
                                     sems[b2], add=True)

                @pl.when(jc + 2 <= _NCHUNK - 1)
                def _():
                    start_idx(jc + 2, b2, (b3 + 2) % 3, with_sh)

            def step(j6, carry):
                for i in range(6):
                    slot(6 * j6 + i, i % 2, i % 3)
                return carry

            lax.fori_loop(0, (_NCHUNK + 5) // 6, step, 0)

            pltpu.make_async_copy(src_of(1), acc.at[didx3[0]],
                                  sems[1]).wait()
            pltpu.make_async_copy(src_of(0), acc.at[didx3[1]],
                                  sems[0]).wait()

            plsc.subcore_barrier()
            _flush_acc(sid, cid, acc, out_hbm)

        _zero_buf(scaled[0], _CB)
        _zero_buf(scaled[1], _CB)
        _zero_acc(sid, scaled[0], acc)
        plsc.subcore_barrier()
        run_phase(a_hbm, p0_hbm, False, None)
        plsc.subcore_barrier()

        _zero_acc(sid, scaled[0], acc)
        plsc.subcore_barrier()

        def computeB(b):
            rb = rows[b]
            sb = scaled[b]
            def grp(g, carry):
                sv = shf[b][pl.ds(16 * g, 16)]
                for i in range(4):
                    e = 4 * g + i
                    r0 = rb[e, 0:16]
                    r1 = rb[e, 16:32]
                    for k in range(3):
                        s = sv[4 * i + k]
                        sb[e, 32 * k:32 * k + 16] = r0 * s
                        sb[e, 32 * k + 16:32 * k + 32] = r1 * s
                return carry
            lax.fori_loop(0, _CB // 4, grp, 0, unroll=10)

        run_phase(b_hbm, p1_hbm, True, computeB)


@jax.jit
def kernel(atom_feature, edge_vector, edge_index, W0, W1):
    src = edge_index[0]
    dst = edge_index[1]
    x2 = edge_vector[:, 0].reshape(_N, 32)
    y2 = edge_vector[:, 1].reshape(_N, 32)
    z2 = edge_vector[:, 2].reshape(_N, 32)
    eye32 = jnp.eye(32, dtype=jnp.float32)
    px = jnp.concatenate(
        [jnp.kron(eye32, jnp.eye(1, 4, k, dtype=jnp.float32))
         for k in range(3)], axis=1)
    gsum = jnp.kron(eye32, jnp.ones((4, 4), jnp.float32))

    B = pl.pallas_call(
        _bmat_body,
        grid=(10,),
        in_specs=[
            pl.BlockSpec((_N // 10, _CIN), lambda i: (i, 0)),
            pl.BlockSpec((_CIN, _CIN), lambda i: (0, 0)),
        ],
        out_specs=pl.BlockSpec((_N // 10, _CIN), lambda i: (i, 0)),
        out_shape=jax.ShapeDtypeStruct((_N, _CIN), jnp.float32),
    )(atom_feature, jnp.pad(W1, ((0, 0), (0, _CIN - _C1))))

    nr = _E * 4 // 128
    sh = pl.pallas_call(
        _sh_body,
        grid=(10,),
        in_specs=[
            pl.BlockSpec((_N // 10, 32), lambda i: (i, 0)),
            pl.BlockSpec((_N // 10, 32), lambda i: (i, 0)),
            pl.BlockSpec((_N // 10, 32), lambda i: (i, 0)),
            pl.BlockSpec((32, 384), lambda i: (0, 0)),
            pl.BlockSpec((128, 128), lambda i: (0, 0)),
        ],
        out_specs=pl.BlockSpec((nr // 10, 128), lambda i: (i, 0)),
        out_shape=jax.ShapeDtypeStruct((nr, 128), jnp.float32),
    )(x2, y2, z2, px, gsum)

    mesh = plsc.VectorSubcoreMesh(core_axis_name="c", subcore_axis_name="s")

    p0, p1 = pl.kernel(
        _seg_body,
        out_type=[
            jax.ShapeDtypeStruct((_NC, _N, _CIN), jnp.float32),
            jax.ShapeDtypeStruct((_NC, _N, _CIN), jnp.float32),
        ],
        mesh=mesh,
        compiler_params=pltpu.CompilerParams(use_tc_tiling_on_sc=True),
        scratch_types=[
            pltpu.VMEM((_CB,), jnp.int32),
            pltpu.VMEM((_CB,), jnp.int32),
            pltpu.VMEM((_CB,), jnp.int32),
            pltpu.VMEM((_CB,), jnp.int32),
            pltpu.VMEM((_CB,), jnp.int32),
            pltpu.VMEM((_CB, _CIN), jnp.float32),
            pltpu.VMEM((_CB, _CIN), jnp.float32),
            pltpu.VMEM((_CB, _CIN), jnp.float32),
            pltpu.VMEM((_CB, _CIN), jnp.float32),
            pltpu.VMEM((_CB * 4 + 16,), jnp.float32),
            pltpu.VMEM((_CB * 4 + 16,), jnp.float32),
            pltpu.VMEM_SHARED((_N, _CIN), jnp.float32),
            pltpu.SemaphoreType.DMA,
            pltpu.SemaphoreType.DMA,
            pltpu.SemaphoreType.DMA,
            pltpu.SemaphoreType.DMA,
            pltpu.SemaphoreType.DMA,
            pltpu.SemaphoreType.DMA,
        ],
    )(atom_feature, B, src, dst, sh.reshape(_E * 4))

    ks = jnp.arange(_V1) % 3
    cs = jnp.arange(_V1) // 3
    perm = jnp.zeros((_CIN, _V1), jnp.float32).at[
        ks * _C1 + cs, jnp.arange(_V1)].set(1.0)

    out0, out1 = pl.pallas_call(
        _final_body,
        grid=(10,),
        in_specs=[
            pl.BlockSpec((_NC, _N // 10, _CIN), lambda i: (0, i, 0)),
            pl.BlockSpec((_NC, _N // 10, _CIN), lambda i: (0, i, 0)),
            pl.BlockSpec((_N // 10, _CIN), lambda i: (i, 0)),
            pl.BlockSpec((_CIN, _CIN), lambda i: (0, 0)),
            pl.BlockSpec((_CIN, _V1), lambda i: (0, 0)),
        ],
        out_specs=[
            pl.BlockSpec((_N // 10, _CIN), lambda i: (i, 0)),
            pl.BlockSpec((_N // 10, _V1), lambda i: (i, 0)),
        ],
        out_shape=[
            jax.ShapeDtypeStruct((_N, _CIN), jnp.float32),
            jax.ShapeDtypeStruct((_N, _V1), jnp.float32),
        ],
    )(p0, p1, atom_feature, W0, perm)

    return jnp.concatenate([out0, out1], axis=1)
